# baseline (speedup 1.0000x reference)
import functools
import jax
import jax.numpy as jnp
from jax import lax
from jax.experimental import pallas as pl
from jax.experimental.pallas import tpu as pltpu
from jax.experimental.pallas import tpu_sc as plsc

NN = 10000
EE = 320000
DD = 128
HH = 16
CC = 40

NCORES = 2
NSUB = 16
NW = NCORES * NSUB

CHUNK = 128
RING = 8
EPT = 10240
EPAD = EPT * NW
NCHUNK = EPT // CHUNK
NG = NCHUNK // RING

DEG_ROWS = 20480
AGG_ROWS = 10240

_MESH = plsc.VectorSubcoreMesh(core_axis_name="c", subcore_axis_name="s")
_SC_PARAMS = pltpu.CompilerParams(use_tc_tiling_on_sc=False)


def _fill_rows(buf, nrows, value):
    def body(i, _):
        buf[i, :] = jnp.full((HH,), value, jnp.float32)
        return 0
    lax.fori_loop(0, nrows, body, 0)


def _zero_table(table, zbuf, sub, rows, sem):
    per = rows // NSUB
    base = sub * per
    descs = [
        pltpu.async_copy(zbuf, table.at[pl.ds(base + i * CHUNK, CHUNK)], sem)
        for i in range(per // CHUNK)
    ]
    for d in descs:
        d.wait()


@functools.partial(
    pl.kernel,
    out_type=jax.ShapeDtypeStruct((NCORES, DEG_ROWS, HH), jnp.float32),
    mesh=_MESH,
    compiler_params=_SC_PARAMS,
    scratch_types=[
        pltpu.VMEM((CHUNK, HH), jnp.float32),
        pltpu.VMEM((CHUNK, HH), jnp.float32),
        pltpu.VMEM((2 * NCHUNK, CHUNK), jnp.int32),
        pltpu.VMEM_SHARED((DEG_ROWS, HH), jnp.float32),
        pltpu.SemaphoreType.DMA,
    ],
)
def _deg_kernel(idx3, out, ones_v, zeros_v, idx_v, table, sem0):
    c = lax.axis_index("c")
    s = lax.axis_index("s")
    wid = s * NCORES + c

    pltpu.sync_copy(idx3.at[wid], idx_v)
    _fill_rows(ones_v, CHUNK, 1.0)
    _fill_rows(zeros_v, CHUNK, 0.0)
    _zero_table(table, zeros_v, s, DEG_ROWS, sem0)
    plsc.subcore_barrier()

    @pl.loop(0, 2 * NG)
    def _(g):
        descs = [
            pltpu.async_copy(ones_v, table.at[idx_v.at[g * RING + b]],
                             sem0, add=True)
            for b in range(RING)
        ]
        for d in descs:
            d.wait()

    plsc.subcore_barrier()
    per = DEG_ROWS // NSUB
    pltpu.sync_copy(table.at[pl.ds(s * per, per)],
                    out.at[c, pl.ds(s * per, per)])


@functools.partial(
    pl.kernel,
    out_type=jax.ShapeDtypeStruct((NCORES, AGG_ROWS, HH), jnp.float32),
    mesh=_MESH,
    compiler_params=_SC_PARAMS,
    scratch_types=[
        pltpu.VMEM((NCHUNK, CHUNK), jnp.int32),
        pltpu.VMEM((NCHUNK + 1, CHUNK), jnp.int32),
        pltpu.VMEM((2, RING, CHUNK, HH), jnp.float32),
        pltpu.VMEM((CHUNK, HH), jnp.float32),
        pltpu.VMEM_SHARED((AGG_ROWS, HH), jnp.float32),
        pltpu.SemaphoreType.DMA((2,)),
        pltpu.SemaphoreType.DMA((2,)),
    ],
)
def _agg_kernel(y, srcp3, dstp3, out, sidx, didx, rows, zeros_v, table,
                gsem, ssem):
    c = lax.axis_index("c")
    s = lax.axis_index("s")
    wid = s * NCORES + c

    pltpu.sync_copy(srcp3.at[wid], sidx)
    pltpu.sync_copy(dstp3.at[wid], didx)
    _fill_rows(zeros_v, CHUNK, 0.0)
    _zero_table(table, zeros_v, s, AGG_ROWS, gsem.at[0])
    plsc.subcore_barrier()

    def fire_gathers(g, p):
        for b in range(RING):
            ch = jnp.minimum(g * RING + b, NCHUNK - 1)
            pltpu.async_copy(y.at[sidx.at[ch]], rows.at[p, b], gsem.at[p])

    def drain_gathers(p):
        for b in range(RING):
            pltpu.make_async_copy(y.at[sidx.at[0]], rows.at[p, b],
                                  gsem.at[p]).wait()

    def fire_scatters(g, p):
        for b in range(RING):
            ch = jnp.minimum(g * RING + b, NCHUNK)
            pltpu.async_copy(rows.at[p, b], table.at[didx.at[ch]],
                             ssem.at[p], add=True)

    def drain_scatters(p):
        for b in range(RING):
            pltpu.make_async_copy(rows.at[p, b], table.at[didx.at[0]],
                                  ssem.at[p]).wait()

    fire_gathers(0, 0)
    fire_scatters(NCHUNK, 1)

    @pl.loop(0, NG)
    def _(g):
        p = g % 2
        q = 1 - p
        drain_gathers(p)
        fire_scatters(g, p)
        drain_scatters(q)
        fire_gathers(g + 1, q)

    drain_gathers(NG % 2)
    drain_scatters(1 - NG % 2)
    plsc.subcore_barrier()
    per = AGG_ROWS // NSUB
    pltpu.sync_copy(table.at[pl.ds(s * per, per)],
                    out.at[c, pl.ds(s * per, per)])


def _dense1_body(x_ref, w_ref, z_ref):
    z_ref[...] = jnp.dot(x_ref[...], w_ref[...],
                         preferred_element_type=jnp.float32)


def _norm_body(deg_ref, z_ref, y_ref, ns_ref, nd_ref):
    dsum = deg_ref[0] + deg_ref[1]
    deg_out = dsum[:NN, 0:1]
    deg_in = dsum[NN:2 * NN, 0:1]
    ns = jnp.where(deg_out > 0, lax.rsqrt(jnp.maximum(deg_out, 1e-12)), 0.0)
    nd = jnp.where(deg_in > 0, lax.rsqrt(jnp.maximum(deg_in, 1e-12)), 0.0)
    ns_ref[...] = ns
    nd_ref[...] = nd
    y_ref[...] = z_ref[...] * ns


def _mid_body(agg_ref, ns_ref, nd_ref, b_ref, w_ref, y_ref):
    aggsum = agg_ref[0, :NN] + agg_ref[1, :NN]
    h = jnp.maximum(aggsum * nd_ref[...] + b_ref[...], 0.0)
    y_ref[...] = jnp.dot(h * ns_ref[...], w_ref[...],
                         preferred_element_type=jnp.float32)


def _premix_body(agg_ref, ns_ref, nd_ref, b_ref, y_ref):
    aggsum = agg_ref[0, :NN] + agg_ref[1, :NN]
    h = jnp.maximum(aggsum * nd_ref[...] + b_ref[...], 0.0)
    y_ref[...] = h * ns_ref[...]


def _final_body(agg_ref, nd_ref, b_ref, w_ref, out_ref):
    aggsum = agg_ref[0, :NN] + agg_ref[1, :NN]
    out_ref[...] = jnp.dot(aggsum * nd_ref[...], w_ref[...],
                           preferred_element_type=jnp.float32) + b_ref[...]


def _tc(body, out_shape):
    return pl.pallas_call(body, out_shape=out_shape)


_f32 = jnp.float32


def kernel(features, edge_index, W1, b1, W2, b2, W3, b3):
    src = edge_index[0]
    dst = edge_index[1]
    npad = EPAD - EE
    srcp = jnp.concatenate([src, jnp.zeros((npad,), jnp.int32)])
    dstp = jnp.concatenate([dst, jnp.full((npad,), NN, jnp.int32)])
    srcd = jnp.concatenate([src, jnp.full((npad,), 2 * NN, jnp.int32)])
    dstd = jnp.concatenate([dst + NN, jnp.full((npad,), 2 * NN, jnp.int32)])
    srcp3 = srcp.reshape(NW, NCHUNK, CHUNK)
    dstp3 = jnp.concatenate([dstp.reshape(NW, NCHUNK, CHUNK),
                             jnp.full((NW, 1, CHUNK), NN, jnp.int32)], axis=1)
    idx3 = jnp.concatenate([srcd.reshape(NW, NCHUNK, CHUNK),
                            dstd.reshape(NW, NCHUNK, CHUNK)], axis=1)

    deg_parts = _deg_kernel(idx3)
    z1 = _tc(_dense1_body, jax.ShapeDtypeStruct((NN, HH), _f32))(features, W1)
    y1, ns, nd = _tc(_norm_body, (jax.ShapeDtypeStruct((NN, HH), _f32),
                                  jax.ShapeDtypeStruct((NN, 1), _f32),
                                  jax.ShapeDtypeStruct((NN, 1), _f32)))(
        deg_parts, z1)

    a1 = _agg_kernel(y1, srcp3, dstp3)
    y2 = _tc(_mid_body, jax.ShapeDtypeStruct((NN, HH), _f32))(
        a1, ns, nd, b1.reshape(1, HH), W2)

    a2 = _agg_kernel(y2, srcp3, dstp3)
    y3 = _tc(_premix_body, jax.ShapeDtypeStruct((NN, HH), _f32))(
        a2, ns, nd, b2.reshape(1, HH))

    a3 = _agg_kernel(y3, srcp3, dstp3)
    out = _tc(_final_body, jax.ShapeDtypeStruct((NN, CC), _f32))(
        a3, nd, b3.reshape(1, CC), W3)
    return out


# speedup vs baseline: 1.2369x; 1.2369x over previous
import functools
import jax
import jax.numpy as jnp
from jax import lax
from jax.experimental import pallas as pl
from jax.experimental.pallas import tpu as pltpu
from jax.experimental.pallas import tpu_sc as plsc

NN = 10000
EE = 320000
DD = 128
HH = 16
CC = 40

NCORES = 2
NSUB = 16
NW = NCORES * NSUB

CHUNK = 128
RING = 8

FAST_CORE = 0
KF_A, KS_A = 112, 48
KF_D, KS_D = 200, 120
POOL_A = NSUB * (KF_A + KS_A)
POOL_D = NSUB * (KF_D + KS_D)
EPAD = POOL_A * CHUNK
NCH_E = EE // CHUNK

DEG_ROWS = 20480
AGG_ROWS = 10240

_MESH = plsc.VectorSubcoreMesh(core_axis_name="c", subcore_axis_name="s")
_SC_PARAMS = pltpu.CompilerParams(use_tc_tiling_on_sc=False)


def _fill_rows(buf, nrows, value):
    def body(i, _):
        buf[i, :] = jnp.full((HH,), value, jnp.float32)
        return 0
    lax.fori_loop(0, nrows, body, 0)


def _zero_table(table, zbuf, sub, rows, sem):
    per = rows // NSUB
    base = sub * per
    descs = [
        pltpu.async_copy(zbuf, table.at[pl.ds(base + i * CHUNK, CHUNK)], sem)
        for i in range(per // CHUNK)
    ]
    for d in descs:
        d.wait()


@functools.partial(
    pl.kernel,
    out_type=jax.ShapeDtypeStruct((NCORES, DEG_ROWS, HH), jnp.float32),
    mesh=_MESH,
    compiler_params=_SC_PARAMS,
    scratch_types=[
        pltpu.VMEM((CHUNK, HH), jnp.float32),
        pltpu.VMEM((CHUNK, HH), jnp.float32),
        pltpu.VMEM((KF_D, CHUNK), jnp.int32),
        pltpu.VMEM_SHARED((DEG_ROWS, HH), jnp.float32),
        pltpu.SemaphoreType.DMA,
    ],
)
def _deg_kernel(idx3, out, ones_v, zeros_v, idx_v, table, sem0):
    c = lax.axis_index("c")
    s = lax.axis_index("s")
    trow = jnp.where(c == FAST_CORE, s, NSUB + s)
    ngrp = jnp.where(c == FAST_CORE, KF_D // RING, KS_D // RING)

    pltpu.sync_copy(idx3.at[trow], idx_v)
    _fill_rows(ones_v, CHUNK, 1.0)
    _fill_rows(zeros_v, CHUNK, 0.0)
    _zero_table(table, zeros_v, s, DEG_ROWS, sem0)
    plsc.subcore_barrier()

    @pl.loop(0, ngrp)
    def _(g):
        descs = [
            pltpu.async_copy(ones_v, table.at[idx_v.at[g * RING + b]],
                             sem0, add=True)
            for b in range(RING)
        ]
        for d in descs:
            d.wait()

    plsc.subcore_barrier()
    per = DEG_ROWS // NSUB
    pltpu.sync_copy(table.at[pl.ds(s * per, per)],
                    out.at[c, pl.ds(s * per, per)])


@functools.partial(
    pl.kernel,
    out_type=jax.ShapeDtypeStruct((NCORES, AGG_ROWS, HH), jnp.float32),
    mesh=_MESH,
    compiler_params=_SC_PARAMS,
    scratch_types=[
        pltpu.VMEM((KF_A, CHUNK), jnp.int32),
        pltpu.VMEM((KF_A, CHUNK), jnp.int32),
        pltpu.VMEM((RING, CHUNK, HH), jnp.float32),
        pltpu.VMEM((CHUNK, HH), jnp.float32),
        pltpu.VMEM_SHARED((AGG_ROWS, HH), jnp.float32),
        pltpu.SemaphoreType.DMA,
        pltpu.SemaphoreType.DMA,
    ],
)
def _agg_kernel(y, srcp3, dstp3, out, sidx, didx, rows, zeros_v, table,
                gsem, ssem):
    c = lax.axis_index("c")
    s = lax.axis_index("s")
    trow = jnp.where(c == FAST_CORE, s, NSUB + s)
    ngrp = jnp.where(c == FAST_CORE, KF_A // RING, KS_A // RING)

    pltpu.sync_copy(srcp3.at[trow], sidx)
    pltpu.sync_copy(dstp3.at[trow], didx)
    _fill_rows(zeros_v, CHUNK, 0.0)
    _zero_table(table, zeros_v, s, AGG_ROWS, gsem)
    plsc.subcore_barrier()

    @pl.loop(0, ngrp)
    def _(g):
        gd = [
            pltpu.async_copy(y.at[sidx.at[g * RING + b]], rows.at[b], gsem)
            for b in range(RING)
        ]
        for d in gd:
            d.wait()
        sd = [
            pltpu.async_copy(rows.at[b], table.at[didx.at[g * RING + b]],
                             ssem, add=True)
            for b in range(RING)
        ]
        for d in sd:
            d.wait()

    plsc.subcore_barrier()
    per = AGG_ROWS // NSUB
    pltpu.sync_copy(table.at[pl.ds(s * per, per)],
                    out.at[c, pl.ds(s * per, per)])


def _dense1_body(x_ref, w_ref, z_ref):
    z_ref[...] = jnp.dot(x_ref[...], w_ref[...],
                         preferred_element_type=jnp.float32)


def _norm_body(deg_ref, z_ref, y_ref, ns_ref, nd_ref):
    dsum = deg_ref[0] + deg_ref[1]
    deg_out = dsum[:NN, 0:1]
    deg_in = dsum[NN:2 * NN, 0:1]
    ns = jnp.where(deg_out > 0, lax.rsqrt(jnp.maximum(deg_out, 1e-12)), 0.0)
    nd = jnp.where(deg_in > 0, lax.rsqrt(jnp.maximum(deg_in, 1e-12)), 0.0)
    ns_ref[...] = ns
    nd_ref[...] = nd
    y_ref[...] = z_ref[...] * ns


def _mid_body(agg_ref, ns_ref, nd_ref, b_ref, w_ref, y_ref):
    aggsum = agg_ref[0, :NN] + agg_ref[1, :NN]
    h = jnp.maximum(aggsum * nd_ref[...] + b_ref[...], 0.0)
    y_ref[...] = jnp.dot(h * ns_ref[...], w_ref[...],
                         preferred_element_type=jnp.float32)


def _premix_body(agg_ref, ns_ref, nd_ref, b_ref, y_ref):
    aggsum = agg_ref[0, :NN] + agg_ref[1, :NN]
    h = jnp.maximum(aggsum * nd_ref[...] + b_ref[...], 0.0)
    y_ref[...] = h * ns_ref[...]


def _final_body(agg_ref, nd_ref, b_ref, w_ref, out_ref):
    aggsum = agg_ref[0, :NN] + agg_ref[1, :NN]
    out_ref[...] = jnp.dot(aggsum * nd_ref[...], w_ref[...],
                           preferred_element_type=jnp.float32) + b_ref[...]


def _tc(body, out_shape):
    return pl.pallas_call(body, out_shape=out_shape)


_f32 = jnp.float32


def _tile_chunks(chunks, kf, ks, padval):
    fast = chunks[:NSUB * kf].reshape(NSUB, kf, CHUNK)
    slow = chunks[NSUB * kf:].reshape(NSUB, ks, CHUNK)
    slow = jnp.concatenate(
        [slow, jnp.full((NSUB, kf - ks, CHUNK), padval, jnp.int32)], axis=1)
    return jnp.concatenate([fast, slow], axis=0)


def kernel(features, edge_index, W1, b1, W2, b2, W3, b3):
    src = edge_index[0]
    dst = edge_index[1]
    npad = EPAD - EE
    srcp = jnp.concatenate([src, jnp.zeros((npad,), jnp.int32)])
    dstp = jnp.concatenate([dst, jnp.full((npad,), NN, jnp.int32)])
    srcp3 = _tile_chunks(srcp.reshape(POOL_A, CHUNK), KF_A, KS_A, 0)
    dstp3 = _tile_chunks(dstp.reshape(POOL_A, CHUNK), KF_A, KS_A, NN)
    degd = jnp.concatenate([
        src.reshape(NCH_E, CHUNK), (dst + NN).reshape(NCH_E, CHUNK),
        jnp.full((POOL_D - 2 * NCH_E, CHUNK), 2 * NN, jnp.int32)])
    idx3 = _tile_chunks(degd, KF_D, KS_D, 2 * NN)

    deg_parts = _deg_kernel(idx3)
    z1 = _tc(_dense1_body, jax.ShapeDtypeStruct((NN, HH), _f32))(features, W1)
    y1, ns, nd = _tc(_norm_body, (jax.ShapeDtypeStruct((NN, HH), _f32),
                                  jax.ShapeDtypeStruct((NN, 1), _f32),
                                  jax.ShapeDtypeStruct((NN, 1), _f32)))(
        deg_parts, z1)

    a1 = _agg_kernel(y1, srcp3, dstp3)
    y2 = _tc(_mid_body, jax.ShapeDtypeStruct((NN, HH), _f32))(
        a1, ns, nd, b1.reshape(1, HH), W2)

    a2 = _agg_kernel(y2, srcp3, dstp3)
    y3 = _tc(_premix_body, jax.ShapeDtypeStruct((NN, HH), _f32))(
        a2, ns, nd, b2.reshape(1, HH))

    a3 = _agg_kernel(y3, srcp3, dstp3)
    out = _tc(_final_body, jax.ShapeDtypeStruct((NN, CC), _f32))(
        a3, nd, b3.reshape(1, CC), W3)
    return out


# speedup vs baseline: 1.3552x; 1.0956x over previous
import functools
import jax
import jax.numpy as jnp
from jax import lax
from jax.experimental import pallas as pl
from jax.experimental.pallas import tpu as pltpu
from jax.experimental.pallas import tpu_sc as plsc

NN = 10000
EE = 320000
DD = 128
HH = 16
CC = 40

NCORES = 2
NSUB = 16
NW = NCORES * NSUB

CHUNK = 128
RING = 8

FAST_CORE = 0
KF_A, KS_A = 136, 24
KF_D, KS_D = 240, 80
POOL_A = NSUB * (KF_A + KS_A)
POOL_D = NSUB * (KF_D + KS_D)
EPAD = POOL_A * CHUNK
NCH_E = EE // CHUNK

DEG_ROWS = 20480
AGG_ROWS = 10240

_MESH = plsc.VectorSubcoreMesh(core_axis_name="c", subcore_axis_name="s")
_SC_PARAMS = pltpu.CompilerParams(use_tc_tiling_on_sc=False)


def _fill_rows(buf, nrows, value):
    def body(i, _):
        buf[i, :] = jnp.full((HH,), value, jnp.float32)
        return 0
    lax.fori_loop(0, nrows, body, 0)


def _zero_table(table, zbuf, sub, rows, sem):
    per = rows // NSUB
    base = sub * per
    descs = [
        pltpu.async_copy(zbuf, table.at[pl.ds(base + i * CHUNK, CHUNK)], sem)
        for i in range(per // CHUNK)
    ]
    for d in descs:
        d.wait()


@functools.partial(
    pl.kernel,
    out_type=jax.ShapeDtypeStruct((NCORES, DEG_ROWS, HH), jnp.float32),
    mesh=_MESH,
    compiler_params=_SC_PARAMS,
    scratch_types=[
        pltpu.VMEM((CHUNK, HH), jnp.float32),
        pltpu.VMEM((CHUNK, HH), jnp.float32),
        pltpu.VMEM((KF_D, CHUNK), jnp.int32),
        pltpu.VMEM_SHARED((DEG_ROWS, HH), jnp.float32),
        pltpu.SemaphoreType.DMA,
    ],
)
def _deg_kernel(idx3, out, ones_v, zeros_v, idx_v, table, sem0):
    c = lax.axis_index("c")
    s = lax.axis_index("s")
    trow = jnp.where(c == FAST_CORE, s, NSUB + s)
    ngrp = jnp.where(c == FAST_CORE, KF_D // RING, KS_D // RING)

    @pl.when(c == FAST_CORE)
    def _():
        pltpu.sync_copy(idx3.at[trow], idx_v)

    @pl.when(c != FAST_CORE)
    def _():
        pltpu.sync_copy(idx3.at[trow, pl.ds(0, KS_D)],
                        idx_v.at[pl.ds(0, KS_D)])
    _fill_rows(ones_v, CHUNK, 1.0)
    _fill_rows(zeros_v, CHUNK, 0.0)
    _zero_table(table, zeros_v, s, DEG_ROWS, sem0)
    plsc.subcore_barrier()

    @pl.loop(0, ngrp)
    def _(g):
        descs = [
            pltpu.async_copy(ones_v, table.at[idx_v.at[g * RING + b]],
                             sem0, add=True)
            for b in range(RING)
        ]
        for d in descs:
            d.wait()

    plsc.subcore_barrier()
    per = DEG_ROWS // NSUB
    pltpu.sync_copy(table.at[pl.ds(s * per, per)],
                    out.at[c, pl.ds(s * per, per)])


@functools.partial(
    pl.kernel,
    out_type=jax.ShapeDtypeStruct((NCORES, AGG_ROWS, HH), jnp.float32),
    mesh=_MESH,
    compiler_params=_SC_PARAMS,
    scratch_types=[
        pltpu.VMEM((KF_A, CHUNK), jnp.int32),
        pltpu.VMEM((KF_A, CHUNK), jnp.int32),
        pltpu.VMEM((RING, CHUNK, HH), jnp.float32),
        pltpu.VMEM((CHUNK, HH), jnp.float32),
        pltpu.VMEM_SHARED((AGG_ROWS, HH), jnp.float32),
        pltpu.SemaphoreType.DMA,
        pltpu.SemaphoreType.DMA,
    ],
)
def _agg_kernel(y, srcp3, dstp3, out, sidx, didx, rows, zeros_v, table,
                gsem, ssem):
    c = lax.axis_index("c")
    s = lax.axis_index("s")
    trow = jnp.where(c == FAST_CORE, s, NSUB + s)
    ngrp = jnp.where(c == FAST_CORE, KF_A // RING, KS_A // RING)

    @pl.when(c == FAST_CORE)
    def _():
        pltpu.sync_copy(srcp3.at[trow], sidx)
        pltpu.sync_copy(dstp3.at[trow], didx)

    @pl.when(c != FAST_CORE)
    def _():
        pltpu.sync_copy(srcp3.at[trow, pl.ds(0, KS_A)],
                        sidx.at[pl.ds(0, KS_A)])
        pltpu.sync_copy(dstp3.at[trow, pl.ds(0, KS_A)],
                        didx.at[pl.ds(0, KS_A)])
    _fill_rows(zeros_v, CHUNK, 0.0)
    _zero_table(table, zeros_v, s, AGG_ROWS, gsem)
    plsc.subcore_barrier()

    @pl.loop(0, ngrp)
    def _(g):
        gd = [
            pltpu.async_copy(y.at[sidx.at[g * RING + b]], rows.at[b], gsem)
            for b in range(RING)
        ]
        for d in gd:
            d.wait()
        sd = [
            pltpu.async_copy(rows.at[b], table.at[didx.at[g * RING + b]],
                             ssem, add=True)
            for b in range(RING)
        ]
        for d in sd:
            d.wait()

    plsc.subcore_barrier()
    per = AGG_ROWS // NSUB
    pltpu.sync_copy(table.at[pl.ds(s * per, per)],
                    out.at[c, pl.ds(s * per, per)])


def _pre_body(deg_ref, x_ref, w_ref, y_ref, ns_ref, nd_ref):
    dsum = deg_ref[0] + deg_ref[1]
    deg_out = dsum[:NN, 0:1]
    deg_in = dsum[NN:2 * NN, 0:1]
    ns = jnp.where(deg_out > 0, lax.rsqrt(jnp.maximum(deg_out, 1e-12)), 0.0)
    nd = jnp.where(deg_in > 0, lax.rsqrt(jnp.maximum(deg_in, 1e-12)), 0.0)
    ns_ref[...] = ns
    nd_ref[...] = nd
    y_ref[...] = jnp.dot(x_ref[...], w_ref[...],
                         preferred_element_type=jnp.float32) * ns


def _mid_body(agg_ref, ns_ref, nd_ref, b_ref, w_ref, y_ref):
    aggsum = agg_ref[0, :NN] + agg_ref[1, :NN]
    h = jnp.maximum(aggsum * nd_ref[...] + b_ref[...], 0.0)
    y_ref[...] = jnp.dot(h * ns_ref[...], w_ref[...],
                         preferred_element_type=jnp.float32)


def _premix_body(agg_ref, ns_ref, nd_ref, b_ref, y_ref):
    aggsum = agg_ref[0, :NN] + agg_ref[1, :NN]
    h = jnp.maximum(aggsum * nd_ref[...] + b_ref[...], 0.0)
    y_ref[...] = h * ns_ref[...]


def _final_body(agg_ref, nd_ref, b_ref, w_ref, out_ref):
    aggsum = agg_ref[0, :NN] + agg_ref[1, :NN]
    out_ref[...] = jnp.dot(aggsum * nd_ref[...], w_ref[...],
                           preferred_element_type=jnp.float32) + b_ref[...]


def _tc(body, out_shape):
    return pl.pallas_call(body, out_shape=out_shape)


_f32 = jnp.float32


def _tile_chunks(chunks, kf, ks, padval):
    fast = chunks[:NSUB * kf].reshape(NSUB, kf, CHUNK)
    slow = chunks[NSUB * kf:].reshape(NSUB, ks, CHUNK)
    slow = jnp.concatenate(
        [slow, jnp.full((NSUB, kf - ks, CHUNK), padval, jnp.int32)], axis=1)
    return jnp.concatenate([fast, slow], axis=0)


def kernel(features, edge_index, W1, b1, W2, b2, W3, b3):
    src = edge_index[0]
    dst = edge_index[1]
    npad = EPAD - EE
    srcp = jnp.concatenate([src, jnp.zeros((npad,), jnp.int32)])
    dstp = jnp.concatenate([dst, jnp.full((npad,), NN, jnp.int32)])
    srcp3 = _tile_chunks(srcp.reshape(POOL_A, CHUNK), KF_A, KS_A, 0)
    dstp3 = _tile_chunks(dstp.reshape(POOL_A, CHUNK), KF_A, KS_A, NN)
    degd = jnp.concatenate([
        src.reshape(NCH_E, CHUNK), (dst + NN).reshape(NCH_E, CHUNK),
        jnp.full((POOL_D - 2 * NCH_E, CHUNK), 2 * NN, jnp.int32)])
    idx3 = _tile_chunks(degd, KF_D, KS_D, 2 * NN)

    deg_parts = _deg_kernel(idx3)
    y1, ns, nd = _tc(_pre_body, (jax.ShapeDtypeStruct((NN, HH), _f32),
                                 jax.ShapeDtypeStruct((NN, 1), _f32),
                                 jax.ShapeDtypeStruct((NN, 1), _f32)))(
        deg_parts, features, W1)

    a1 = _agg_kernel(y1, srcp3, dstp3)
    y2 = _tc(_mid_body, jax.ShapeDtypeStruct((NN, HH), _f32))(
        a1, ns, nd, b1.reshape(1, HH), W2)

    a2 = _agg_kernel(y2, srcp3, dstp3)
    y3 = _tc(_premix_body, jax.ShapeDtypeStruct((NN, HH), _f32))(
        a2, ns, nd, b2.reshape(1, HH))

    a3 = _agg_kernel(y3, srcp3, dstp3)
    out = _tc(_final_body, jax.ShapeDtypeStruct((NN, CC), _f32))(
        a3, nd, b3.reshape(1, CC), W3)
    return out


# speedup vs baseline: 1.4687x; 1.0837x over previous
import functools
import jax
import jax.numpy as jnp
from jax import lax
from jax.experimental import pallas as pl
from jax.experimental.pallas import tpu as pltpu
from jax.experimental.pallas import tpu_sc as plsc

NN = 10000
EE = 320000
DD = 128
HH = 16
CC = 40

NCORES = 2
NSUB = 16
NW = NCORES * NSUB

CHUNK = 128
RING = 8
RING_D = 4
NCH_E = EE // CHUNK

FAST_CORE = 0
KF_A = 136
CS_A = NCH_E - NSUB * KF_A
CSB_A, CSR_A = CS_A // NSUB, CS_A % NSUB
KSPRE_A = 32

KF_P = 120
CS_P = NCH_E - NSUB * KF_P
CSB_P, CSR_P = CS_P // NSUB, CS_P % NSUB
KSPRE_P = 48

AGG_ROWS = 10240

_MESH = plsc.VectorSubcoreMesh(core_axis_name="c", subcore_axis_name="s")
_SC_PARAMS = pltpu.CompilerParams(use_tc_tiling_on_sc=False)


def _fill_rows(buf, nrows, value):
    def body(i, _):
        buf[i, :] = jnp.full((HH,), value, jnp.float32)
        return 0
    lax.fori_loop(0, nrows, body, 0)


def _zero_tables(tables, zbuf, sub, sem):
    per = AGG_ROWS // NSUB
    base = sub * per
    descs = [
        pltpu.async_copy(zbuf, t.at[pl.ds(base + i * CHUNK, CHUNK)], sem)
        for t in tables
        for i in range(per // CHUNK)
    ]
    for d in descs:
        d.wait()


def _split(s, is_fast, kf, csb, csr, kspre):
    start = jnp.where(is_fast, kf * s,
                      NSUB * kf + csb * s + jnp.minimum(s, csr))
    nc = jnp.where(is_fast, kf, csb + (s < csr).astype(jnp.int32))
    pre = jnp.where(is_fast, start, jnp.minimum(start, NCH_E - kspre))
    return pre, start - pre, nc


@functools.partial(
    pl.kernel,
    out_type=jax.ShapeDtypeStruct((NCORES, 2, AGG_ROWS, HH), jnp.float32),
    mesh=_MESH,
    compiler_params=_SC_PARAMS,
    scratch_types=[
        pltpu.VMEM((CHUNK, HH), jnp.float32),
        pltpu.VMEM((CHUNK, HH), jnp.float32),
        pltpu.VMEM((KF_P, CHUNK), jnp.int32),
        pltpu.VMEM((KF_P, CHUNK), jnp.int32),
        pltpu.VMEM_SHARED((AGG_ROWS, HH), jnp.float32),
        pltpu.VMEM_SHARED((AGG_ROWS, HH), jnp.float32),
        pltpu.SemaphoreType.DMA,
    ],
)
def _deg_kernel(eidx3, out, ones_v, zeros_v, sidx, didx, tout, tin, sem):
    c = lax.axis_index("c")
    s = lax.axis_index("s")
    is_fast = c == FAST_CORE
    pre, offs, nc = _split(s, is_fast, KF_P, CSB_P, CSR_P, KSPRE_P)

    @pl.when(is_fast)
    def _():
        pltpu.sync_copy(eidx3.at[0, pl.ds(pre, KF_P)], sidx)
        pltpu.sync_copy(eidx3.at[1, pl.ds(pre, KF_P)], didx)

    @pl.when(jnp.logical_not(is_fast))
    def _():
        pltpu.sync_copy(eidx3.at[0, pl.ds(pre, KSPRE_P)],
                        sidx.at[pl.ds(0, KSPRE_P)])
        pltpu.sync_copy(eidx3.at[1, pl.ds(pre, KSPRE_P)],
                        didx.at[pl.ds(0, KSPRE_P)])

    _fill_rows(ones_v, CHUNK, 1.0)
    _fill_rows(zeros_v, CHUNK, 0.0)
    _zero_tables((tout, tin), zeros_v, s, sem)
    plsc.subcore_barrier()

    @pl.loop(0, nc // RING_D)
    def _(g):
        descs = []
        for b in range(RING_D):
            ch = offs + g * RING_D + b
            descs.append(pltpu.async_copy(ones_v, tout.at[sidx.at[ch]],
                                          sem, add=True))
            descs.append(pltpu.async_copy(ones_v, tin.at[didx.at[ch]],
                                          sem, add=True))
        for d in descs:
            d.wait()

    @pl.loop(0, nc % RING_D)
    def _(i):
        ch = offs + (nc // RING_D) * RING_D + i
        d0 = pltpu.async_copy(ones_v, tout.at[sidx.at[ch]], sem, add=True)
        d1 = pltpu.async_copy(ones_v, tin.at[didx.at[ch]], sem, add=True)
        d0.wait()
        d1.wait()

    plsc.subcore_barrier()
    per = AGG_ROWS // NSUB
    pltpu.sync_copy(tout.at[pl.ds(s * per, per)],
                    out.at[c, 0, pl.ds(s * per, per)])
    pltpu.sync_copy(tin.at[pl.ds(s * per, per)],
                    out.at[c, 1, pl.ds(s * per, per)])


@functools.partial(
    pl.kernel,
    out_type=jax.ShapeDtypeStruct((NCORES, AGG_ROWS, HH), jnp.float32),
    mesh=_MESH,
    compiler_params=_SC_PARAMS,
    scratch_types=[
        pltpu.VMEM((KF_A, CHUNK), jnp.int32),
        pltpu.VMEM((KF_A, CHUNK), jnp.int32),
        pltpu.VMEM((RING, CHUNK, HH), jnp.float32),
        pltpu.VMEM((CHUNK, HH), jnp.float32),
        pltpu.VMEM_SHARED((AGG_ROWS, HH), jnp.float32),
        pltpu.SemaphoreType.DMA,
        pltpu.SemaphoreType.DMA,
    ],
)
def _agg_kernel(y, eidx3, out, sidx, didx, rows, zeros_v, table, gsem, ssem):
    c = lax.axis_index("c")
    s = lax.axis_index("s")
    is_fast = c == FAST_CORE
    pre, offs, nc = _split(s, is_fast, KF_A, CSB_A, CSR_A, KSPRE_A)

    @pl.when(is_fast)
    def _():
        pltpu.sync_copy(eidx3.at[0, pl.ds(pre, KF_A)], sidx)
        pltpu.sync_copy(eidx3.at[1, pl.ds(pre, KF_A)], didx)

    @pl.when(jnp.logical_not(is_fast))
    def _():
        pltpu.sync_copy(eidx3.at[0, pl.ds(pre, KSPRE_A)],
                        sidx.at[pl.ds(0, KSPRE_A)])
        pltpu.sync_copy(eidx3.at[1, pl.ds(pre, KSPRE_A)],
                        didx.at[pl.ds(0, KSPRE_A)])

    _fill_rows(zeros_v, CHUNK, 0.0)
    _zero_tables((table,), zeros_v, s, gsem)
    plsc.subcore_barrier()

    @pl.loop(0, nc // RING)
    def _(g):
        gd = [
            pltpu.async_copy(y.at[sidx.at[offs + g * RING + b]],
                             rows.at[b], gsem)
            for b in range(RING)
        ]
        for d in gd:
            d.wait()
        sd = [
            pltpu.async_copy(rows.at[b], table.at[didx.at[offs + g * RING + b]],
                             ssem, add=True)
            for b in range(RING)
        ]
        for d in sd:
            d.wait()

    @pl.loop(0, nc % RING)
    def _(i):
        ch = offs + (nc // RING) * RING + i
        pltpu.async_copy(y.at[sidx.at[ch]], rows.at[0], gsem).wait()
        pltpu.async_copy(rows.at[0], table.at[didx.at[ch]],
                         ssem, add=True).wait()

    plsc.subcore_barrier()
    per = AGG_ROWS // NSUB
    pltpu.sync_copy(table.at[pl.ds(s * per, per)],
                    out.at[c, pl.ds(s * per, per)])


def _pre_body(deg_ref, x_ref, w_ref, y_ref, ns_ref, nd_ref):
    deg_out = deg_ref[0, 0, :NN, 0:1] + deg_ref[1, 0, :NN, 0:1]
    deg_in = deg_ref[0, 1, :NN, 0:1] + deg_ref[1, 1, :NN, 0:1]
    ns = jnp.where(deg_out > 0, lax.rsqrt(jnp.maximum(deg_out, 1e-12)), 0.0)
    nd = jnp.where(deg_in > 0, lax.rsqrt(jnp.maximum(deg_in, 1e-12)), 0.0)
    ns_ref[...] = ns
    nd_ref[...] = nd
    y_ref[...] = jnp.dot(x_ref[...], w_ref[...],
                         preferred_element_type=jnp.float32) * ns


def _mid_body(agg_ref, ns_ref, nd_ref, b_ref, w_ref, y_ref):
    aggsum = agg_ref[0, :NN] + agg_ref[1, :NN]
    h = jnp.maximum(aggsum * nd_ref[...] + b_ref[...], 0.0)
    y_ref[...] = jnp.dot(h * ns_ref[...], w_ref[...],
                         preferred_element_type=jnp.float32)


def _premix_body(agg_ref, ns_ref, nd_ref, b_ref, y_ref):
    aggsum = agg_ref[0, :NN] + agg_ref[1, :NN]
    h = jnp.maximum(aggsum * nd_ref[...] + b_ref[...], 0.0)
    y_ref[...] = h * ns_ref[...]


def _final_body(agg_ref, nd_ref, b_ref, w_ref, out_ref):
    aggsum = agg_ref[0, :NN] + agg_ref[1, :NN]
    out_ref[...] = jnp.dot(aggsum * nd_ref[...], w_ref[...],
                           preferred_element_type=jnp.float32) + b_ref[...]


def _tc(body, out_shape):
    return pl.pallas_call(body, out_shape=out_shape)


_f32 = jnp.float32


def kernel(features, edge_index, W1, b1, W2, b2, W3, b3):
    eidx3 = edge_index.reshape(2, NCH_E, CHUNK)

    deg_parts = _deg_kernel(eidx3)
    y1, ns, nd = _tc(_pre_body, (jax.ShapeDtypeStruct((NN, HH), _f32),
                                 jax.ShapeDtypeStruct((NN, 1), _f32),
                                 jax.ShapeDtypeStruct((NN, 1), _f32)))(
        deg_parts, features, W1)

    a1 = _agg_kernel(y1, eidx3)
    y2 = _tc(_mid_body, jax.ShapeDtypeStruct((NN, HH), _f32))(
        a1, ns, nd, b1.reshape(1, HH), W2)

    a2 = _agg_kernel(y2, eidx3)
    y3 = _tc(_premix_body, jax.ShapeDtypeStruct((NN, HH), _f32))(
        a2, ns, nd, b2.reshape(1, HH))

    a3 = _agg_kernel(y3, eidx3)
    out = _tc(_final_body, jax.ShapeDtypeStruct((NN, CC), _f32))(
        a3, nd, b3.reshape(1, CC), W3)
    return out


# speedup vs baseline: 1.6953x; 1.1543x over previous
import functools
import jax
import jax.numpy as jnp
from jax import lax
from jax.experimental import pallas as pl
from jax.experimental.pallas import tpu as pltpu
from jax.experimental.pallas import tpu_sc as plsc

NN = 10000
EE = 320000
DD = 128
HH = 16
CC = 40

NCORES = 2
NSUB = 16
NW = NCORES * NSUB

CHUNK = 128
RING = 8
RING_D = 4
NCH_E = EE // CHUNK

FAST_CORE = 0
KF_A = 96
CS_A = NCH_E - NSUB * KF_A
CSB_A, CSR_A = CS_A // NSUB, CS_A % NSUB
KSPRE_A = 80

KF_P = 104
CS_P = NCH_E - NSUB * KF_P
CSB_P, CSR_P = CS_P // NSUB, CS_P % NSUB
KSPRE_P = 64

AGG_ROWS = 10240

_MESH = plsc.VectorSubcoreMesh(core_axis_name="c", subcore_axis_name="s")
_SC_PARAMS = pltpu.CompilerParams(use_tc_tiling_on_sc=False)


def _fill_rows(buf, nrows, value):
    def body(i, _):
        buf[i, :] = jnp.full((HH,), value, jnp.float32)
        return 0
    lax.fori_loop(0, nrows, body, 0)


def _zero_tables(tables, zbuf, sub, sem):
    per = AGG_ROWS // NSUB
    base = sub * per
    descs = [
        pltpu.async_copy(zbuf, t.at[pl.ds(base + i * CHUNK, CHUNK)], sem)
        for t in tables
        for i in range(per // CHUNK)
    ]
    for d in descs:
        d.wait()


def _split(s, is_fast, kf, csb, csr, kspre):
    start = jnp.where(is_fast, kf * s,
                      NSUB * kf + csb * s + jnp.minimum(s, csr))
    nc = jnp.where(is_fast, kf, csb + (s < csr).astype(jnp.int32))
    pre = jnp.where(is_fast, start, jnp.minimum(start, NCH_E - kspre))
    return pre, start - pre, nc


@functools.partial(
    pl.kernel,
    out_type=jax.ShapeDtypeStruct((NCORES, 2, AGG_ROWS, HH), jnp.float32),
    mesh=_MESH,
    compiler_params=_SC_PARAMS,
    scratch_types=[
        pltpu.VMEM((CHUNK, HH), jnp.float32),
        pltpu.VMEM((CHUNK, HH), jnp.float32),
        pltpu.VMEM((KF_P, CHUNK), jnp.int32),
        pltpu.VMEM((KF_P, CHUNK), jnp.int32),
        pltpu.VMEM_SHARED((AGG_ROWS, HH), jnp.float32),
        pltpu.VMEM_SHARED((AGG_ROWS, HH), jnp.float32),
        pltpu.SemaphoreType.DMA,
    ],
)
def _deg_kernel(eidx3, out, ones_v, zeros_v, sidx, didx, tout, tin, sem):
    c = lax.axis_index("c")
    s = lax.axis_index("s")
    is_fast = c == FAST_CORE
    pre, offs, nc = _split(s, is_fast, KF_P, CSB_P, CSR_P, KSPRE_P)

    @pl.when(is_fast)
    def _():
        pltpu.sync_copy(eidx3.at[0, pl.ds(pre, KF_P)], sidx)
        pltpu.sync_copy(eidx3.at[1, pl.ds(pre, KF_P)], didx)

    @pl.when(jnp.logical_not(is_fast))
    def _():
        pltpu.sync_copy(eidx3.at[0, pl.ds(pre, KSPRE_P)],
                        sidx.at[pl.ds(0, KSPRE_P)])
        pltpu.sync_copy(eidx3.at[1, pl.ds(pre, KSPRE_P)],
                        didx.at[pl.ds(0, KSPRE_P)])

    _fill_rows(ones_v, CHUNK, 1.0)
    _fill_rows(zeros_v, CHUNK, 0.0)
    _zero_tables((tout, tin), zeros_v, s, sem)
    plsc.subcore_barrier()

    @pl.loop(0, nc // RING_D)
    def _(g):
        descs = []
        for b in range(RING_D):
            ch = offs + g * RING_D + b
            descs.append(pltpu.async_copy(ones_v, tout.at[sidx.at[ch]],
                                          sem, add=True))
            descs.append(pltpu.async_copy(ones_v, tin.at[didx.at[ch]],
                                          sem, add=True))
        for d in descs:
            d.wait()

    @pl.loop(0, nc % RING_D)
    def _(i):
        ch = offs + (nc // RING_D) * RING_D + i
        d0 = pltpu.async_copy(ones_v, tout.at[sidx.at[ch]], sem, add=True)
        d1 = pltpu.async_copy(ones_v, tin.at[didx.at[ch]], sem, add=True)
        d0.wait()
        d1.wait()

    plsc.subcore_barrier()
    per = AGG_ROWS // NSUB
    pltpu.sync_copy(tout.at[pl.ds(s * per, per)],
                    out.at[c, 0, pl.ds(s * per, per)])
    pltpu.sync_copy(tin.at[pl.ds(s * per, per)],
                    out.at[c, 1, pl.ds(s * per, per)])


@functools.partial(
    pl.kernel,
    out_type=jax.ShapeDtypeStruct((NCORES, AGG_ROWS, HH), jnp.float32),
    mesh=_MESH,
    compiler_params=_SC_PARAMS,
    scratch_types=[
        pltpu.VMEM((KF_A, CHUNK), jnp.int32),
        pltpu.VMEM((KF_A, CHUNK), jnp.int32),
        pltpu.VMEM((RING, CHUNK, HH), jnp.float32),
        pltpu.VMEM((CHUNK, HH), jnp.float32),
        pltpu.VMEM_SHARED((AGG_ROWS, HH), jnp.float32),
        pltpu.SemaphoreType.DMA,
        pltpu.SemaphoreType.DMA,
    ],
)
def _agg_kernel(y, eidx3, out, sidx, didx, rows, zeros_v, table, gsem, ssem):
    c = lax.axis_index("c")
    s = lax.axis_index("s")
    is_fast = c == FAST_CORE
    pre, offs, nc = _split(s, is_fast, KF_A, CSB_A, CSR_A, KSPRE_A)

    @pl.when(is_fast)
    def _():
        pltpu.sync_copy(eidx3.at[0, pl.ds(pre, KF_A)], sidx)
        pltpu.sync_copy(eidx3.at[1, pl.ds(pre, KF_A)], didx)

    @pl.when(jnp.logical_not(is_fast))
    def _():
        pltpu.sync_copy(eidx3.at[0, pl.ds(pre, KSPRE_A)],
                        sidx.at[pl.ds(0, KSPRE_A)])
        pltpu.sync_copy(eidx3.at[1, pl.ds(pre, KSPRE_A)],
                        didx.at[pl.ds(0, KSPRE_A)])

    _fill_rows(zeros_v, CHUNK, 0.0)
    _zero_tables((table,), zeros_v, s, gsem)
    plsc.subcore_barrier()

    @pl.loop(0, nc // RING)
    def _(g):
        gd = [
            pltpu.async_copy(y.at[sidx.at[offs + g * RING + b]],
                             rows.at[b], gsem)
            for b in range(RING)
        ]
        for d in gd:
            d.wait()
        sd = [
            pltpu.async_copy(rows.at[b], table.at[didx.at[offs + g * RING + b]],
                             ssem, add=True)
            for b in range(RING)
        ]
        for d in sd:
            d.wait()

    @pl.loop(0, nc % RING)
    def _(i):
        ch = offs + (nc // RING) * RING + i
        pltpu.async_copy(y.at[sidx.at[ch]], rows.at[0], gsem).wait()
        pltpu.async_copy(rows.at[0], table.at[didx.at[ch]],
                         ssem, add=True).wait()

    plsc.subcore_barrier()
    per = AGG_ROWS // NSUB
    pltpu.sync_copy(table.at[pl.ds(s * per, per)],
                    out.at[c, pl.ds(s * per, per)])


def _pre_body(deg_ref, x_ref, w_ref, y_ref, ns_ref, nd_ref):
    deg_out = deg_ref[0, 0, :NN, 0:1] + deg_ref[1, 0, :NN, 0:1]
    deg_in = deg_ref[0, 1, :NN, 0:1] + deg_ref[1, 1, :NN, 0:1]
    ns = jnp.where(deg_out > 0, lax.rsqrt(jnp.maximum(deg_out, 1e-12)), 0.0)
    nd = jnp.where(deg_in > 0, lax.rsqrt(jnp.maximum(deg_in, 1e-12)), 0.0)
    ns_ref[...] = ns
    nd_ref[...] = nd
    y_ref[...] = jnp.dot(x_ref[...], w_ref[...],
                         preferred_element_type=jnp.float32) * ns


def _mid_body(agg_ref, ns_ref, nd_ref, b_ref, w_ref, y_ref):
    aggsum = agg_ref[0, :NN] + agg_ref[1, :NN]
    h = jnp.maximum(aggsum * nd_ref[...] + b_ref[...], 0.0)
    y_ref[...] = jnp.dot(h * ns_ref[...], w_ref[...],
                         preferred_element_type=jnp.float32)


def _premix_body(agg_ref, ns_ref, nd_ref, b_ref, y_ref):
    aggsum = agg_ref[0, :NN] + agg_ref[1, :NN]
    h = jnp.maximum(aggsum * nd_ref[...] + b_ref[...], 0.0)
    y_ref[...] = h * ns_ref[...]


def _final_body(agg_ref, nd_ref, b_ref, w_ref, out_ref):
    aggsum = agg_ref[0, :NN] + agg_ref[1, :NN]
    out_ref[...] = jnp.dot(aggsum * nd_ref[...], w_ref[...],
                           preferred_element_type=jnp.float32) + b_ref[...]


def _tc(body, out_shape):
    return pl.pallas_call(body, out_shape=out_shape)


_f32 = jnp.float32


def kernel(features, edge_index, W1, b1, W2, b2, W3, b3):
    eidx3 = edge_index.reshape(2, NCH_E, CHUNK)

    deg_parts = _deg_kernel(eidx3)
    y1, ns, nd = _tc(_pre_body, (jax.ShapeDtypeStruct((NN, HH), _f32),
                                 jax.ShapeDtypeStruct((NN, 1), _f32),
                                 jax.ShapeDtypeStruct((NN, 1), _f32)))(
        deg_parts, features, W1)

    a1 = _agg_kernel(y1, eidx3)
    y2 = _tc(_mid_body, jax.ShapeDtypeStruct((NN, HH), _f32))(
        a1, ns, nd, b1.reshape(1, HH), W2)

    a2 = _agg_kernel(y2, eidx3)
    y3 = _tc(_premix_body, jax.ShapeDtypeStruct((NN, HH), _f32))(
        a2, ns, nd, b2.reshape(1, HH))

    a3 = _agg_kernel(y3, eidx3)
    out = _tc(_final_body, jax.ShapeDtypeStruct((NN, CC), _f32))(
        a3, nd, b3.reshape(1, CC), W3)
    return out


# speedup vs baseline: 2.5649x; 1.5130x over previous
import functools
import jax
import jax.numpy as jnp
from jax import lax
from jax.experimental import pallas as pl
from jax.experimental.pallas import tpu as pltpu
from jax.experimental.pallas import tpu_sc as plsc

NN = 10000
EE = 320000
DD = 128
HH = 16
CC = 40

NCORES = 2
NSUB = 16
NW = NCORES * NSUB

CHUNK = 128
RING = 8
RING_D = 4
NCH_E = EE // CHUNK

FAST_CORE = 0
KF_A = 84
CS_A = NCH_E - NSUB * KF_A
CSB_A, CSR_A = CS_A // NSUB, CS_A % NSUB
KSPRE_A = 96

KF_P = 80
CS_P = NCH_E - NSUB * KF_P
CSB_P, CSR_P = CS_P // NSUB, CS_P % NSUB
KSPRE_P = 96
SROWS = 96

AGG_ROWS = 10240

_MESH = plsc.VectorSubcoreMesh(core_axis_name="c", subcore_axis_name="s")
_SC_PARAMS = pltpu.CompilerParams(use_tc_tiling_on_sc=False)


def _fill_rows(buf, nrows, value):
    def body(i, _):
        buf[i, :] = jnp.full((HH,), value, jnp.float32)
        return 0
    lax.fori_loop(0, nrows, body, 0)


def _zero_tables(tables, zbuf, sub, sem):
    per = AGG_ROWS // NSUB
    base = sub * per
    descs = [
        pltpu.async_copy(zbuf, t.at[pl.ds(base + i * CHUNK, CHUNK)], sem)
        for t in tables
        for i in range(per // CHUNK)
    ]
    for d in descs:
        d.wait()


def _split(s, is_fast, kf, csb, csr, kspre):
    start = jnp.where(is_fast, kf * s,
                      NSUB * kf + csb * s + jnp.minimum(s, csr))
    nc = jnp.where(is_fast, kf, csb + (s < csr).astype(jnp.int32))
    pre = jnp.where(is_fast, start, jnp.minimum(start, NCH_E - kspre))
    return pre, start - pre, nc


@functools.partial(
    pl.kernel,
    out_type=jax.ShapeDtypeStruct((NCORES, 2, AGG_ROWS, HH), jnp.float32),
    mesh=_MESH,
    compiler_params=_SC_PARAMS,
    scratch_types=[
        pltpu.VMEM((CHUNK, HH), jnp.float32),
        pltpu.VMEM((CHUNK, HH), jnp.float32),
        pltpu.VMEM((SROWS, CHUNK), jnp.int32),
        pltpu.VMEM((SROWS, CHUNK), jnp.int32),
        pltpu.VMEM_SHARED((AGG_ROWS, HH), jnp.float32),
        pltpu.VMEM_SHARED((AGG_ROWS, HH), jnp.float32),
        pltpu.SemaphoreType.DMA,
    ],
)
def _deg_kernel(eidx3, out, ones_v, zeros_v, sidx, didx, tout, tin, sem):
    c = lax.axis_index("c")
    s = lax.axis_index("s")
    is_fast = c == FAST_CORE
    pre, offs, nc = _split(s, is_fast, KF_P, CSB_P, CSR_P, KSPRE_P)

    @pl.when(is_fast)
    def _():
        pltpu.sync_copy(eidx3.at[0, pl.ds(pre, KF_P)],
                        sidx.at[pl.ds(0, KF_P)])
        pltpu.sync_copy(eidx3.at[1, pl.ds(pre, KF_P)],
                        didx.at[pl.ds(0, KF_P)])

    @pl.when(jnp.logical_not(is_fast))
    def _():
        pltpu.sync_copy(eidx3.at[0, pl.ds(pre, KSPRE_P)],
                        sidx.at[pl.ds(0, KSPRE_P)])
        pltpu.sync_copy(eidx3.at[1, pl.ds(pre, KSPRE_P)],
                        didx.at[pl.ds(0, KSPRE_P)])

    _fill_rows(ones_v, CHUNK, 1.0)
    _fill_rows(zeros_v, CHUNK, 0.0)
    _zero_tables((tout, tin), zeros_v, s, sem)
    plsc.subcore_barrier()

    @pl.loop(0, nc // RING_D)
    def _(g):
        descs = []
        for b in range(RING_D):
            ch = offs + g * RING_D + b
            descs.append(pltpu.async_copy(ones_v, tout.at[sidx.at[ch]],
                                          sem, add=True))
            descs.append(pltpu.async_copy(ones_v, tin.at[didx.at[ch]],
                                          sem, add=True))
        for d in descs:
            d.wait()

    @pl.loop(0, nc % RING_D)
    def _(i):
        ch = offs + (nc // RING_D) * RING_D + i
        d0 = pltpu.async_copy(ones_v, tout.at[sidx.at[ch]], sem, add=True)
        d1 = pltpu.async_copy(ones_v, tin.at[didx.at[ch]], sem, add=True)
        d0.wait()
        d1.wait()

    plsc.subcore_barrier()
    per = AGG_ROWS // NSUB
    pltpu.sync_copy(tout.at[pl.ds(s * per, per)],
                    out.at[c, 0, pl.ds(s * per, per)])
    pltpu.sync_copy(tin.at[pl.ds(s * per, per)],
                    out.at[c, 1, pl.ds(s * per, per)])


@functools.partial(
    pl.kernel,
    out_type=jax.ShapeDtypeStruct((NCORES, AGG_ROWS, HH), jnp.float32),
    mesh=_MESH,
    compiler_params=_SC_PARAMS,
    scratch_types=[
        pltpu.VMEM((SROWS, CHUNK), jnp.int32),
        pltpu.VMEM((SROWS, CHUNK), jnp.int32),
        pltpu.VMEM((RING, CHUNK, HH), jnp.float32),
        pltpu.VMEM((CHUNK, HH), jnp.float32),
        pltpu.VMEM_SHARED((AGG_ROWS, HH), jnp.float32),
        pltpu.SemaphoreType.DMA,
        pltpu.SemaphoreType.DMA,
    ],
)
def _agg_kernel(y, eidx3, out, sidx, didx, rows, zeros_v, table, gsem, ssem):
    c = lax.axis_index("c")
    s = lax.axis_index("s")
    is_fast = c == FAST_CORE
    pre, offs, nc = _split(s, is_fast, KF_A, CSB_A, CSR_A, KSPRE_A)

    @pl.when(is_fast)
    def _():
        pltpu.sync_copy(eidx3.at[0, pl.ds(pre, KF_A)],
                        sidx.at[pl.ds(0, KF_A)])
        pltpu.sync_copy(eidx3.at[1, pl.ds(pre, KF_A)],
                        didx.at[pl.ds(0, KF_A)])

    @pl.when(jnp.logical_not(is_fast))
    def _():
        pltpu.sync_copy(eidx3.at[0, pl.ds(pre, KSPRE_A)],
                        sidx.at[pl.ds(0, KSPRE_A)])
        pltpu.sync_copy(eidx3.at[1, pl.ds(pre, KSPRE_A)],
                        didx.at[pl.ds(0, KSPRE_A)])

    _fill_rows(zeros_v, CHUNK, 0.0)
    _zero_tables((table,), zeros_v, s, gsem)
    plsc.subcore_barrier()

    @pl.loop(0, nc // RING)
    def _(g):
        gd = [
            pltpu.async_copy(y.at[sidx.at[offs + g * RING + b]],
                             rows.at[b], gsem)
            for b in range(RING)
        ]
        for d in gd:
            d.wait()
        sd = [
            pltpu.async_copy(rows.at[b], table.at[didx.at[offs + g * RING + b]],
                             ssem, add=True)
            for b in range(RING)
        ]
        for d in sd:
            d.wait()

    @pl.loop(0, nc % RING)
    def _(i):
        ch = offs + (nc // RING) * RING + i
        pltpu.async_copy(y.at[sidx.at[ch]], rows.at[0], gsem).wait()
        pltpu.async_copy(rows.at[0], table.at[didx.at[ch]],
                         ssem, add=True).wait()

    plsc.subcore_barrier()
    per = AGG_ROWS // NSUB
    pltpu.sync_copy(table.at[pl.ds(s * per, per)],
                    out.at[c, pl.ds(s * per, per)])


NR = NN // 8
TR = AGG_ROWS // 8


def _pre_body(deg_ref, x_ref, w_ref, y_ref, ns_ref, nd_ref):
    deg_out = deg_ref[0, 0, :NR] + deg_ref[1, 0, :NR]
    deg_in = deg_ref[0, 1, :NR] + deg_ref[1, 1, :NR]
    ns = jnp.where(deg_out > 0, lax.rsqrt(jnp.maximum(deg_out, 1e-12)), 0.0)
    nd = jnp.where(deg_in > 0, lax.rsqrt(jnp.maximum(deg_in, 1e-12)), 0.0)
    ns_ref[...] = ns
    nd_ref[...] = nd
    y_ref[...] = jnp.dot(x_ref[...], w_ref[...],
                         preferred_element_type=jnp.float32) * ns


def _mid_body(agg_ref, ns_ref, nd_ref, b_ref, w_ref, y_ref):
    aggsum = agg_ref[0, :NR] + agg_ref[1, :NR]
    h = jnp.maximum(aggsum * nd_ref[...] + b_ref[...], 0.0)
    y_ref[...] = jnp.dot(h * ns_ref[...], w_ref[...],
                         preferred_element_type=jnp.float32)


def _premix_body(agg_ref, ns_ref, nd_ref, b_ref, y_ref):
    aggsum = agg_ref[0, :NR] + agg_ref[1, :NR]
    h = jnp.maximum(aggsum * nd_ref[...] + b_ref[...], 0.0)
    y_ref[...] = h * ns_ref[...]


def _final_body(agg_ref, nd_ref, b_ref, w_ref, out_ref):
    aggsum = agg_ref[0, :NR] + agg_ref[1, :NR]
    out_ref[...] = jnp.dot(aggsum * nd_ref[...], w_ref[...],
                           preferred_element_type=jnp.float32) + b_ref[...]


def _tc(body, out_shape):
    return pl.pallas_call(body, out_shape=out_shape)


_f32 = jnp.float32


def kernel(features, edge_index, W1, b1, W2, b2, W3, b3):
    eidx3 = edge_index.reshape(2, NCH_E, CHUNK)
    eye8 = jnp.eye(8, dtype=_f32)
    x8 = features.reshape(NR, 8 * DD)
    bd1 = jnp.kron(eye8, W1)
    bd2 = jnp.kron(eye8, W2)
    bd3 = jnp.kron(eye8, W3)
    b1r = jnp.tile(b1, 8).reshape(1, 128)
    b2r = jnp.tile(b2, 8).reshape(1, 128)
    b3r = jnp.tile(b3, 8).reshape(1, 8 * CC)

    deg128 = _deg_kernel(eidx3).reshape(NCORES, 2, TR, 128)
    y1, ns, nd = _tc(_pre_body, (jax.ShapeDtypeStruct((NR, 128), _f32),
                                 jax.ShapeDtypeStruct((NR, 128), _f32),
                                 jax.ShapeDtypeStruct((NR, 128), _f32)))(
        deg128, x8, bd1)

    a1 = _agg_kernel(y1.reshape(NN, HH), eidx3).reshape(NCORES, TR, 128)
    y2 = _tc(_mid_body, jax.ShapeDtypeStruct((NR, 128), _f32))(
        a1, ns, nd, b1r, bd2)

    a2 = _agg_kernel(y2.reshape(NN, HH), eidx3).reshape(NCORES, TR, 128)
    y3 = _tc(_premix_body, jax.ShapeDtypeStruct((NR, 128), _f32))(
        a2, ns, nd, b2r)

    a3 = _agg_kernel(y3.reshape(NN, HH), eidx3).reshape(NCORES, TR, 128)
    out = _tc(_final_body, jax.ShapeDtypeStruct((NR, 8 * CC), _f32))(
        a3, nd, b3r, bd3)
    return out.reshape(NN, CC)


# speedup vs baseline: 2.7982x; 1.0910x over previous
import functools
import jax
import jax.numpy as jnp
from jax import lax
from jax.experimental import pallas as pl
from jax.experimental.pallas import tpu as pltpu
from jax.experimental.pallas import tpu_sc as plsc

NN = 10000
EE = 320000
DD = 128
HH = 16
CC = 40

NCORES = 2
NSUB = 16
NW = NCORES * NSUB

CHUNK = 128
RING = 8
RING_D = 4
NCH_E = EE // CHUNK

FAST_CORE = 0
KF_A = 76
CS_A = NCH_E - NSUB * KF_A
CSB_A, CSR_A = CS_A // NSUB, CS_A % NSUB
KSPRE_A = 96

KF_P = 80
CS_P = NCH_E - NSUB * KF_P
CSB_P, CSR_P = CS_P // NSUB, CS_P % NSUB
KSPRE_P = 96
SROWS = 96

AGG_ROWS = 10240

_MESH = plsc.VectorSubcoreMesh(core_axis_name="c", subcore_axis_name="s")
_SC_PARAMS = pltpu.CompilerParams(use_tc_tiling_on_sc=False)


def _fill_rows(buf, nrows, value):
    def body(i, _):
        buf[i, :] = jnp.full((HH,), value, jnp.float32)
        return 0
    lax.fori_loop(0, nrows, body, 0)


def _zero_tables(tables, zbuf, sub, sem):
    per = AGG_ROWS // NSUB
    base = sub * per
    descs = [
        pltpu.async_copy(zbuf, t.at[pl.ds(base + i * CHUNK, CHUNK)], sem)
        for t in tables
        for i in range(per // CHUNK)
    ]
    for d in descs:
        d.wait()


def _split(s, is_fast, kf, csb, csr, kspre):
    start = jnp.where(is_fast, kf * s,
                      NSUB * kf + csb * s + jnp.minimum(s, csr))
    nc = jnp.where(is_fast, kf, csb + (s < csr).astype(jnp.int32))
    pre = jnp.where(is_fast, start, jnp.minimum(start, NCH_E - kspre))
    return pre, start - pre, nc


@functools.partial(
    pl.kernel,
    out_type=jax.ShapeDtypeStruct((NCORES, 2, AGG_ROWS, HH), jnp.float32),
    mesh=_MESH,
    compiler_params=_SC_PARAMS,
    scratch_types=[
        pltpu.VMEM((CHUNK, HH), jnp.float32),
        pltpu.VMEM((CHUNK, HH), jnp.float32),
        pltpu.VMEM((SROWS, CHUNK), jnp.int32),
        pltpu.VMEM((SROWS, CHUNK), jnp.int32),
        pltpu.VMEM_SHARED((AGG_ROWS, HH), jnp.float32),
        pltpu.VMEM_SHARED((AGG_ROWS, HH), jnp.float32),
        pltpu.SemaphoreType.DMA,
    ],
)
def _deg_kernel(eidx3, out, ones_v, zeros_v, sidx, didx, tout, tin, sem):
    c = lax.axis_index("c")
    s = lax.axis_index("s")
    is_fast = c == FAST_CORE
    pre, offs, nc = _split(s, is_fast, KF_P, CSB_P, CSR_P, KSPRE_P)

    @pl.when(is_fast)
    def _():
        pltpu.sync_copy(eidx3.at[0, pl.ds(pre, KF_P)],
                        sidx.at[pl.ds(0, KF_P)])
        pltpu.sync_copy(eidx3.at[1, pl.ds(pre, KF_P)],
                        didx.at[pl.ds(0, KF_P)])

    @pl.when(jnp.logical_not(is_fast))
    def _():
        pltpu.sync_copy(eidx3.at[0, pl.ds(pre, KSPRE_P)],
                        sidx.at[pl.ds(0, KSPRE_P)])
        pltpu.sync_copy(eidx3.at[1, pl.ds(pre, KSPRE_P)],
                        didx.at[pl.ds(0, KSPRE_P)])

    _fill_rows(ones_v, CHUNK, 1.0)
    _fill_rows(zeros_v, CHUNK, 0.0)
    _zero_tables((tout, tin), zeros_v, s, sem)
    plsc.subcore_barrier()

    @pl.loop(0, nc // RING_D)
    def _(g):
        descs = []
        for b in range(RING_D):
            ch = offs + g * RING_D + b
            descs.append(pltpu.async_copy(ones_v, tout.at[sidx.at[ch]],
                                          sem, add=True))
            descs.append(pltpu.async_copy(ones_v, tin.at[didx.at[ch]],
                                          sem, add=True))
        for d in descs:
            d.wait()

    @pl.loop(0, nc % RING_D)
    def _(i):
        ch = offs + (nc // RING_D) * RING_D + i
        d0 = pltpu.async_copy(ones_v, tout.at[sidx.at[ch]], sem, add=True)
        d1 = pltpu.async_copy(ones_v, tin.at[didx.at[ch]], sem, add=True)
        d0.wait()
        d1.wait()

    plsc.subcore_barrier()
    per = AGG_ROWS // NSUB
    pltpu.sync_copy(tout.at[pl.ds(s * per, per)],
                    out.at[c, 0, pl.ds(s * per, per)])
    pltpu.sync_copy(tin.at[pl.ds(s * per, per)],
                    out.at[c, 1, pl.ds(s * per, per)])


@functools.partial(
    pl.kernel,
    out_type=jax.ShapeDtypeStruct((NCORES, AGG_ROWS, HH), jnp.float32),
    mesh=_MESH,
    compiler_params=_SC_PARAMS,
    scratch_types=[
        pltpu.VMEM((SROWS, CHUNK), jnp.int32),
        pltpu.VMEM((SROWS, CHUNK), jnp.int32),
        pltpu.VMEM((RING, CHUNK, HH), jnp.float32),
        pltpu.VMEM((CHUNK, HH), jnp.float32),
        pltpu.VMEM_SHARED((AGG_ROWS, HH), jnp.float32),
        pltpu.SemaphoreType.DMA,
        pltpu.SemaphoreType.DMA,
    ],
)
def _agg_kernel(y, eidx3, out, sidx, didx, rows, zeros_v, table, gsem, ssem):
    c = lax.axis_index("c")
    s = lax.axis_index("s")
    is_fast = c == FAST_CORE
    pre, offs, nc = _split(s, is_fast, KF_A, CSB_A, CSR_A, KSPRE_A)

    @pl.when(is_fast)
    def _():
        pltpu.sync_copy(eidx3.at[0, pl.ds(pre, KF_A)],
                        sidx.at[pl.ds(0, KF_A)])
        pltpu.sync_copy(eidx3.at[1, pl.ds(pre, KF_A)],
                        didx.at[pl.ds(0, KF_A)])

    @pl.when(jnp.logical_not(is_fast))
    def _():
        pltpu.sync_copy(eidx3.at[0, pl.ds(pre, KSPRE_A)],
                        sidx.at[pl.ds(0, KSPRE_A)])
        pltpu.sync_copy(eidx3.at[1, pl.ds(pre, KSPRE_A)],
                        didx.at[pl.ds(0, KSPRE_A)])

    _fill_rows(zeros_v, CHUNK, 0.0)
    _zero_tables((table,), zeros_v, s, gsem)
    plsc.subcore_barrier()

    HALF = RING // 2

    @pl.loop(0, nc // RING)
    def _(g):
        gd = [
            pltpu.async_copy(y.at[sidx.at[offs + g * RING + b]],
                             rows.at[b], gsem)
            for b in range(RING)
        ]
        sd = []
        for d in gd[:HALF]:
            d.wait()
        for b in range(HALF):
            sd.append(pltpu.async_copy(
                rows.at[b], table.at[didx.at[offs + g * RING + b]],
                ssem, add=True))
        for d in gd[HALF:]:
            d.wait()
        for b in range(HALF, RING):
            sd.append(pltpu.async_copy(
                rows.at[b], table.at[didx.at[offs + g * RING + b]],
                ssem, add=True))
        for d in sd:
            d.wait()

    @pl.loop(0, nc % RING)
    def _(i):
        ch = offs + (nc // RING) * RING + i
        pltpu.async_copy(y.at[sidx.at[ch]], rows.at[0], gsem).wait()
        pltpu.async_copy(rows.at[0], table.at[didx.at[ch]],
                         ssem, add=True).wait()

    plsc.subcore_barrier()
    per = AGG_ROWS // NSUB
    pltpu.sync_copy(table.at[pl.ds(s * per, per)],
                    out.at[c, pl.ds(s * per, per)])


NR = NN // 8
TR = AGG_ROWS // 8


def _pre_body(deg_ref, x_ref, w_ref, y_ref, ns_ref, nd_ref):
    deg_out = deg_ref[0, 0, :NR] + deg_ref[1, 0, :NR]
    deg_in = deg_ref[0, 1, :NR] + deg_ref[1, 1, :NR]
    ns = jnp.where(deg_out > 0, lax.rsqrt(jnp.maximum(deg_out, 1e-12)), 0.0)
    nd = jnp.where(deg_in > 0, lax.rsqrt(jnp.maximum(deg_in, 1e-12)), 0.0)
    ns_ref[...] = ns
    nd_ref[...] = nd
    y_ref[...] = jnp.dot(x_ref[...], w_ref[...],
                         preferred_element_type=jnp.float32) * ns


def _mid_body(agg_ref, ns_ref, nd_ref, b_ref, w_ref, y_ref):
    aggsum = agg_ref[0, :NR] + agg_ref[1, :NR]
    h = jnp.maximum(aggsum * nd_ref[...] + b_ref[...], 0.0)
    y_ref[...] = jnp.dot(h * ns_ref[...], w_ref[...],
                         preferred_element_type=jnp.float32)


def _premix_body(agg_ref, ns_ref, nd_ref, b_ref, y_ref):
    aggsum = agg_ref[0, :NR] + agg_ref[1, :NR]
    h = jnp.maximum(aggsum * nd_ref[...] + b_ref[...], 0.0)
    y_ref[...] = h * ns_ref[...]


def _final_body(agg_ref, nd_ref, b_ref, w_ref, out_ref):
    aggsum = agg_ref[0, :NR] + agg_ref[1, :NR]
    out_ref[...] = jnp.dot(aggsum * nd_ref[...], w_ref[...],
                           preferred_element_type=jnp.float32) + b_ref[...]


def _tc(body, out_shape):
    return pl.pallas_call(body, out_shape=out_shape)


_f32 = jnp.float32


def kernel(features, edge_index, W1, b1, W2, b2, W3, b3):
    eidx3 = edge_index.reshape(2, NCH_E, CHUNK)
    eye8 = jnp.eye(8, dtype=_f32)
    x8 = features.reshape(NR, 8 * DD)
    bd1 = jnp.kron(eye8, W1)
    bd2 = jnp.kron(eye8, W2)
    bd3 = jnp.kron(eye8, W3)
    b1r = jnp.tile(b1, 8).reshape(1, 128)
    b2r = jnp.tile(b2, 8).reshape(1, 128)
    b3r = jnp.tile(b3, 8).reshape(1, 8 * CC)

    deg128 = _deg_kernel(eidx3).reshape(NCORES, 2, TR, 128)
    y1, ns, nd = _tc(_pre_body, (jax.ShapeDtypeStruct((NR, 128), _f32),
                                 jax.ShapeDtypeStruct((NR, 128), _f32),
                                 jax.ShapeDtypeStruct((NR, 128), _f32)))(
        deg128, x8, bd1)

    a1 = _agg_kernel(y1.reshape(NN, HH), eidx3).reshape(NCORES, TR, 128)
    y2 = _tc(_mid_body, jax.ShapeDtypeStruct((NR, 128), _f32))(
        a1, ns, nd, b1r, bd2)

    a2 = _agg_kernel(y2.reshape(NN, HH), eidx3).reshape(NCORES, TR, 128)
    y3 = _tc(_premix_body, jax.ShapeDtypeStruct((NR, 128), _f32))(
        a2, ns, nd, b2r)

    a3 = _agg_kernel(y3.reshape(NN, HH), eidx3).reshape(NCORES, TR, 128)
    out = _tc(_final_body, jax.ShapeDtypeStruct((NR, 8 * CC), _f32))(
        a3, nd, b3r, bd3)
    return out.reshape(NN, CC)
